# baseline (speedup 1.0000x reference)
import functools
import jax
import jax.numpy as jnp
from jax import lax
from jax.experimental import pallas as pl
from jax.experimental.pallas import tpu as pltpu
from jax.experimental.pallas import tpu_sc as plsc

N = 10000
E = 320000
D = 128
H = 8
C = 8
OUT = 16
G = 64

NP = 10112
NW = 32
BE = 64
NCH = 160
NPAIR = NCH // 2
EPAD = NW * NCH * BE
RPW = NP // 16

_mesh = plsc.VectorSubcoreMesh(core_axis_name="c", subcore_axis_name="s")


def _lrelu(x):
    return jnp.maximum(x, 0.2 * x)


def _pre1_body(x_ref, w1_ref, as_ref, ad_ref, t_ref, m_ref):
    h = jnp.dot(x_ref[...], w1_ref[...], preferred_element_type=jnp.float32)
    a_s = jnp.dot(h, as_ref[...], preferred_element_type=jnp.float32)
    a_d = jnp.dot(h, ad_ref[...], preferred_element_type=jnp.float32)
    t = jnp.concatenate([h, a_s, a_s, a_d, a_d,
                         jnp.zeros((N, 32), jnp.float32)], axis=1)
    t_ref[0:N, :] = t
    t_ref[N:NP, :] = jnp.zeros((NP - N, 128), jnp.float32)
    m8 = _lrelu(jnp.max(a_s, axis=0) + jnp.max(a_d, axis=0))
    m_ref[...] = jnp.concatenate([m8, m8])[None, :]


def _pre1(x, W1cm, Ascm, Adcm):
    return pl.pallas_call(
        _pre1_body,
        out_shape=[
            jax.ShapeDtypeStruct((NP, 128), jnp.float32),
            jax.ShapeDtypeStruct((1, 16), jnp.float32),
        ],
    )(x, W1cm, Ascm, Adcm)


def _make_edge_pass(name, a_src_off, a_dst_off, nblk):
    kernel_kwargs = dict(
        mesh=_mesh,
        out_type=jax.ShapeDtypeStruct((2, NP, 128), jnp.float32),
        scratch_types=[
            pltpu.VMEM((2, BE), jnp.int32),
            pltpu.VMEM((2, BE), jnp.int32),
            pltpu.VMEM((BE, 128), jnp.float32),
            pltpu.VMEM((BE, 128), jnp.float32),
            pltpu.VMEM((BE, 128), jnp.float32),
            pltpu.VMEM((BE, 128), jnp.float32),
            pltpu.VMEM((BE, 128), jnp.float32),
            pltpu.VMEM((16,), jnp.float32),
            pltpu.VMEM_SHARED((NP, 128), jnp.float32),
            pltpu.SemaphoreType.DMA,
            pltpu.SemaphoreType.DMA,
        ],
    )

    def _edge(tbl, sd, mvec, out, sdA, sdB, srowsA, drowsA, srowsB, drowsB,
              msg, mv, acc, semA, semB):
        c = lax.axis_index("c")
        s = lax.axis_index("s")
        wid = s * 2 + c

        pltpu.sync_copy(mvec.at[0], mv)

        def _zrow(i, _):
            zv = jnp.zeros((16,), jnp.float32)
            for k in range(8):
                msg[i, pl.ds(16 * k, 16)] = zv
            return 0

        lax.fori_loop(0, BE, _zrow, 0)
        base = s * RPW
        for st in list(range(0, RPW - BE, BE)) + [RPW - BE]:
            pltpu.sync_copy(msg, acc.at[pl.ds(base + st, BE)])
        plsc.subcore_barrier()

        def _fire(j, sdb, srows, drows, sem):
            pltpu.sync_copy(sd.at[wid, j], sdb)
            pltpu.async_copy(tbl.at[sdb.at[0]], srows, sem)
            pltpu.async_copy(tbl.at[sdb.at[1]], drows, sem)

        def _drain(sdb, srows, drows, sem):
            pltpu.make_async_copy(tbl.at[sdb.at[0]], srows, sem).wait()
            pltpu.make_async_copy(tbl.at[sdb.at[1]], drows, sem).wait()

        def _compute(sdb, srows, drows):
            mreg = mv[...]

            def _edge_i(b, _):
                e = srows[b, pl.ds(a_src_off, 16)] + drows[b, pl.ds(a_dst_off, 16)]
                w = jnp.exp(_lrelu(e) - mreg)
                for k in range(nblk):
                    msg[b, pl.ds(16 * k, 16)] = srows[b, pl.ds(16 * k, 16)] * w
                msg[b, pl.ds(16 * nblk, 16)] = w
                return 0

            lax.fori_loop(0, BE, _edge_i, 0)
            pltpu.sync_copy(msg, acc.at[sdb.at[1]], add=True)

        _fire(0, sdA, srowsA, drowsA, semA)

        def _pair(jp, _):
            j0 = 2 * jp
            _fire(j0 + 1, sdB, srowsB, drowsB, semB)
            _drain(sdA, srowsA, drowsA, semA)
            _compute(sdA, srowsA, drowsA)
            _fire(jnp.minimum(j0 + 2, NCH - 1), sdA, srowsA, drowsA, semA)
            _drain(sdB, srowsB, drowsB, semB)
            _compute(sdB, srowsB, drowsB)
            return 0

        lax.fori_loop(0, NPAIR, _pair, 0)
        _drain(sdA, srowsA, drowsA, semA)
        plsc.subcore_barrier()
        pltpu.sync_copy(acc.at[pl.ds(base, RPW)], out.at[c, pl.ds(base, RPW)])

    _edge.__name__ = name
    _edge.__qualname__ = name
    return pl.kernel(_edge, **kernel_kwargs)


_edge1 = _make_edge_pass("_edge1", 64, 80, 4)
_edge2 = _make_edge_pass("_edge2", 16, 32, 1)


def _mid_body(p_ref, t1_ref, m1_ref, b1_ref, w2_ref, as2_ref,
              ad2_ref, r_ref, t2_ref, m2_ref):
    Rcm = r_ref[...]
    num = p_ref[0:NP, 0:64] + p_ref[NP:2 * NP, 0:64]
    den8 = p_ref[0:NP, 64:72] + p_ref[NP:2 * NP, 64:72]
    h1 = t1_ref[:, 0:64]
    a_s1 = t1_ref[:, 64:72]
    a_d1 = t1_ref[:, 80:88]
    m1 = m1_ref[0:1, 0:8]
    w_self = jnp.exp(_lrelu(a_s1 + a_d1) - m1)
    num = num + jnp.dot(w_self, Rcm, preferred_element_type=jnp.float32) * h1
    den = jnp.dot(den8 + w_self, Rcm, preferred_element_type=jnp.float32)
    o1 = num / (den + 1e-16) + b1_ref[...]
    o1 = jnp.where(o1 > 0, o1, jnp.exp(jnp.minimum(o1, 0.0)) - 1.0)
    rows = lax.broadcasted_iota(jnp.int32, (NP, 64), 0)
    o1 = jnp.where(rows < N, o1, 0.0)
    h2 = jnp.dot(o1, w2_ref[...], preferred_element_type=jnp.float32)
    asc = jnp.dot(h2, as2_ref[...], preferred_element_type=jnp.float32)
    adc = jnp.dot(h2, ad2_ref[...], preferred_element_type=jnp.float32)
    t2_ref[...] = jnp.concatenate(
        [h2, jnp.broadcast_to(asc, (NP, 16)), jnp.broadcast_to(adc, (NP, 16)),
         jnp.zeros((NP, 80), jnp.float32)], axis=1)
    m2 = _lrelu(jnp.max(asc) + jnp.max(adc))
    m2_ref[...] = jnp.full((1, 16), m2, jnp.float32)


def _mid(p1, t1, m1, b1cm, W2cm, as2t, ad2t, Rcm):
    return pl.pallas_call(
        _mid_body,
        out_shape=[
            jax.ShapeDtypeStruct((NP, 128), jnp.float32),
            jax.ShapeDtypeStruct((1, 16), jnp.float32),
        ],
    )(p1, t1, m1, b1cm, W2cm, as2t, ad2t, Rcm)


def _post_body(p_ref, t2_ref, m2_ref, b2_ref, batch_ref, out_ref):
    num2 = p_ref[0:NP, 0:16] + p_ref[NP:2 * NP, 0:16]
    den2 = p_ref[0:NP, 16:17] + p_ref[NP:2 * NP, 16:17]
    h2 = t2_ref[:, 0:16]
    asc = t2_ref[:, 16:17]
    adc = t2_ref[:, 32:33]
    w2 = jnp.exp(_lrelu(asc + adc) - m2_ref[0:1, 0:1])
    numf = num2 + w2 * h2
    denf = den2 + w2
    hout = numf / (denf + 1e-16) + b2_ref[...]
    hN = hout[0:N, :]
    cols = lax.broadcasted_iota(jnp.int32, (N, G), 1)
    onehot = (batch_ref[...] == cols).astype(jnp.float32)
    psum = lax.dot_general(onehot, hN, (((0,), (0,)), ((), ())),
                           preferred_element_type=jnp.float32)
    cnt = lax.dot_general(onehot, jnp.ones((N, 1), jnp.float32),
                          (((0,), (0,)), ((), ())),
                          preferred_element_type=jnp.float32)
    out_ref[...] = psum / jnp.maximum(cnt, 1.0)


def _post(p2, t2, m2, b2, batch2d):
    return pl.pallas_call(
        _post_body,
        out_shape=jax.ShapeDtypeStruct((G, OUT), jnp.float32),
    )(p2, t2, m2, b2, batch2d)


@jax.jit
def kernel(x, edge_index, batch, W1, att_src1, att_dst1, b1,
           W2, att_src2, att_dst2, b2):
    x = x.astype(jnp.float32)
    eye = jnp.eye(H, dtype=jnp.float32)
    W1cm = W1.reshape(D, H, C).transpose(0, 2, 1).reshape(D, H * C)
    Ascm = (att_src1.T[:, :, None] * eye[None, :, :]).reshape(C * H, H)
    Adcm = (att_dst1.T[:, :, None] * eye[None, :, :]).reshape(C * H, H)
    Rcm = jnp.tile(eye, (1, C))
    b1cm = b1.reshape(H, C).T.reshape(1, H * C)
    W2cm = W2.reshape(H, C, OUT).transpose(1, 0, 2).reshape(H * C, OUT)
    pad = jnp.full((EPAD - E,), N, jnp.int32)
    srcs = jnp.concatenate([edge_index[0].astype(jnp.int32), pad])
    dsts = jnp.concatenate([edge_index[1].astype(jnp.int32), pad])
    sd = jnp.stack([srcs.reshape(NW, NCH, BE),
                    dsts.reshape(NW, NCH, BE)], axis=2)

    t1, m1 = _pre1(x, W1cm, Ascm, Adcm)
    p1 = _edge1(t1, sd, m1)
    t2, m2 = _mid(p1.reshape(2 * NP, 128), t1, m1,
                  b1cm, W2cm, att_src2.T, att_dst2.T, Rcm)
    p2 = _edge2(t2, sd, m2)
    pooled = _post(p2.reshape(2 * NP, 128), t2, m2, b2[None, :],
                   batch[:, None].astype(jnp.int32))
    return pooled


# speedup vs baseline: 1.2487x; 1.2487x over previous
import functools
import jax
import jax.numpy as jnp
from jax import lax
from jax.experimental import pallas as pl
from jax.experimental.pallas import tpu as pltpu
from jax.experimental.pallas import tpu_sc as plsc

N = 10000
E = 320000
D = 128
H = 8
C = 8
OUT = 16
G = 64

NP = 10112
NW = 32
BE = 64
NCH = 160
NPAIR = NCH // 2
EPAD = NW * NCH * BE
RPW = NP // 16

_mesh = plsc.VectorSubcoreMesh(core_axis_name="c", subcore_axis_name="s")


def _lrelu(x):
    return jnp.maximum(x, 0.2 * x)


def _pre1_body(x_ref, w1_ref, as_ref, ad_ref, t_ref, td_ref, m_ref):
    h = jnp.dot(x_ref[...], w1_ref[...], preferred_element_type=jnp.float32)
    a_s = jnp.dot(h, as_ref[...], preferred_element_type=jnp.float32)
    a_d = jnp.dot(h, ad_ref[...], preferred_element_type=jnp.float32)
    t = jnp.concatenate([h, a_s, a_s,
                         jnp.zeros((N, 48), jnp.float32)], axis=1)
    t_ref[0:N, :] = t
    t_ref[N:NP, :] = jnp.zeros((NP - N, 128), jnp.float32)
    td_ref[0:N, :] = jnp.concatenate([a_d, a_d,
                                      jnp.zeros((N, 112), jnp.float32)], axis=1)
    td_ref[N:NP, :] = jnp.zeros((NP - N, 128), jnp.float32)
    m8 = _lrelu(jnp.max(a_s, axis=0) + jnp.max(a_d, axis=0))
    m_ref[...] = jnp.concatenate([m8, m8])[None, :]


def _pre1(x, W1cm, Ascm, Adcm):
    return pl.pallas_call(
        _pre1_body,
        out_shape=[
            jax.ShapeDtypeStruct((NP, 128), jnp.float32),
            jax.ShapeDtypeStruct((NP, 128), jnp.float32),
            jax.ShapeDtypeStruct((1, 16), jnp.float32),
        ],
    )(x, W1cm, Ascm, Adcm)


def _make_edge_pass(name, a_src_off, nblk):
    aw = 128
    kernel_kwargs = dict(
        mesh=_mesh,
        out_type=jax.ShapeDtypeStruct((2, NP, aw), jnp.float32),
        scratch_types=[
            pltpu.VMEM((2, BE), jnp.int32),
            pltpu.VMEM((2, BE), jnp.int32),
            pltpu.VMEM((BE, 128), jnp.float32),
            pltpu.VMEM((BE, 128), jnp.float32),
            pltpu.VMEM((BE, 128), jnp.float32),
            pltpu.VMEM((BE, 128), jnp.float32),
            pltpu.VMEM((BE, aw), jnp.float32),
            pltpu.VMEM((16,), jnp.float32),
            pltpu.VMEM_SHARED((NP, aw), jnp.float32),
            pltpu.SemaphoreType.DMA,
            pltpu.SemaphoreType.DMA,
        ],
    )

    def _edge(tsrc, tdst, sd, mvec, out, sdA, sdB, srowsA, drowsA, srowsB,
              drowsB, msg, mv, acc, semA, semB):
        c = lax.axis_index("c")
        s = lax.axis_index("s")
        wid = s * 2 + c

        pltpu.sync_copy(mvec.at[0], mv)

        def _zrow(i, _):
            zv = jnp.zeros((16,), jnp.float32)
            for k in range(8):
                msg[i, pl.ds(16 * k, 16)] = zv
            return 0

        lax.fori_loop(0, BE, _zrow, 0)
        base = s * RPW
        for st in list(range(0, RPW - BE, BE)) + [RPW - BE]:
            pltpu.sync_copy(msg, acc.at[pl.ds(base + st, BE)])
        plsc.subcore_barrier()

        def _fire(j, sdb, srows, drows, sem):
            pltpu.sync_copy(sd.at[wid, j], sdb)
            pltpu.async_copy(tsrc.at[sdb.at[0]], srows, sem)
            pltpu.async_copy(tdst.at[sdb.at[1]], drows, sem)

        def _drain(sdb, srows, drows, sem):
            pltpu.make_async_copy(tsrc.at[sdb.at[0]], srows, sem).wait()
            pltpu.make_async_copy(tdst.at[sdb.at[1]], drows, sem).wait()

        def _compute(sdb, srows, drows):
            mreg = mv[...]

            def _edge_i(b, _):
                e = srows[b, pl.ds(a_src_off, 16)] + drows[b, pl.ds(0, 16)]
                w = jnp.exp(_lrelu(e) - mreg)
                for k in range(nblk):
                    msg[b, pl.ds(16 * k, 16)] = srows[b, pl.ds(16 * k, 16)] * w
                msg[b, pl.ds(16 * nblk, 16)] = w
                return 0

            lax.fori_loop(0, BE, _edge_i, 0)
            pltpu.sync_copy(msg, acc.at[sdb.at[1]], add=True)

        _fire(0, sdA, srowsA, drowsA, semA)

        def _pair(jp, _):
            j0 = 2 * jp
            _fire(j0 + 1, sdB, srowsB, drowsB, semB)
            _drain(sdA, srowsA, drowsA, semA)
            _compute(sdA, srowsA, drowsA)
            _fire(jnp.minimum(j0 + 2, NCH - 1), sdA, srowsA, drowsA, semA)
            _drain(sdB, srowsB, drowsB, semB)
            _compute(sdB, srowsB, drowsB)
            return 0

        lax.fori_loop(0, NPAIR, _pair, 0)
        _drain(sdA, srowsA, drowsA, semA)
        plsc.subcore_barrier()
        pltpu.sync_copy(acc.at[pl.ds(base, RPW)], out.at[c, pl.ds(base, RPW)])

    _edge.__name__ = name
    _edge.__qualname__ = name
    return pl.kernel(_edge, **kernel_kwargs)


_edge1 = _make_edge_pass("_edge1", 64, 4)
_edge2 = _make_edge_pass("_edge2", 16, 1)


def _mid_body(p_ref, t1_ref, td1_ref, m1_ref, b1_ref, w2_ref, as2_ref,
              ad2_ref, r_ref, t2_ref, td2_ref, m2_ref):
    Rcm = r_ref[...]
    num = p_ref[0:NP, 0:64] + p_ref[NP:2 * NP, 0:64]
    den8 = p_ref[0:NP, 64:72] + p_ref[NP:2 * NP, 64:72]
    h1 = t1_ref[:, 0:64]
    a_s1 = t1_ref[:, 64:72]
    a_d1 = td1_ref[:, 0:8]
    m1 = m1_ref[0:1, 0:8]
    w_self = jnp.exp(_lrelu(a_s1 + a_d1) - m1)
    num = num + jnp.dot(w_self, Rcm, preferred_element_type=jnp.float32) * h1
    den = jnp.dot(den8 + w_self, Rcm, preferred_element_type=jnp.float32)
    o1 = num / (den + 1e-16) + b1_ref[...]
    o1 = jnp.where(o1 > 0, o1, jnp.exp(jnp.minimum(o1, 0.0)) - 1.0)
    rows = lax.broadcasted_iota(jnp.int32, (NP, 64), 0)
    o1 = jnp.where(rows < N, o1, 0.0)
    h2 = jnp.dot(o1, w2_ref[...], preferred_element_type=jnp.float32)
    asc = jnp.dot(h2, as2_ref[...], preferred_element_type=jnp.float32)
    adc = jnp.dot(h2, ad2_ref[...], preferred_element_type=jnp.float32)
    t2_ref[...] = jnp.concatenate(
        [h2, jnp.broadcast_to(asc, (NP, 16)), jnp.broadcast_to(adc, (NP, 16)),
         jnp.zeros((NP, 80), jnp.float32)], axis=1)
    td2_ref[...] = jnp.concatenate(
        [jnp.broadcast_to(adc, (NP, 16)), jnp.zeros((NP, 112), jnp.float32)],
        axis=1)
    m2 = _lrelu(jnp.max(asc) + jnp.max(adc))
    m2_ref[...] = jnp.full((1, 16), m2, jnp.float32)


def _mid(p1, t1, td1, m1, b1cm, W2cm, as2t, ad2t, Rcm):
    return pl.pallas_call(
        _mid_body,
        out_shape=[
            jax.ShapeDtypeStruct((NP, 128), jnp.float32),
            jax.ShapeDtypeStruct((NP, 128), jnp.float32),
            jax.ShapeDtypeStruct((1, 16), jnp.float32),
        ],
    )(p1, t1, td1, m1, b1cm, W2cm, as2t, ad2t, Rcm)


def _post_body(p_ref, t2_ref, m2_ref, b2_ref, batch_ref, out_ref):
    num2 = p_ref[0:NP, 0:16] + p_ref[NP:2 * NP, 0:16]
    den2 = p_ref[0:NP, 16:17] + p_ref[NP:2 * NP, 16:17]
    h2 = t2_ref[:, 0:16]
    asc = t2_ref[:, 16:17]
    adc = t2_ref[:, 32:33]
    w2 = jnp.exp(_lrelu(asc + adc) - m2_ref[0:1, 0:1])
    numf = num2 + w2 * h2
    denf = den2 + w2
    hout = numf / (denf + 1e-16) + b2_ref[...]
    hN = hout[0:N, :]
    cols = lax.broadcasted_iota(jnp.int32, (N, G), 1)
    onehot = (batch_ref[...] == cols).astype(jnp.float32)
    psum = lax.dot_general(onehot, hN, (((0,), (0,)), ((), ())),
                           preferred_element_type=jnp.float32)
    cnt = lax.dot_general(onehot, jnp.ones((N, 1), jnp.float32),
                          (((0,), (0,)), ((), ())),
                          preferred_element_type=jnp.float32)
    out_ref[...] = psum / jnp.maximum(cnt, 1.0)


def _post(p2, t2, m2, b2, batch2d):
    return pl.pallas_call(
        _post_body,
        out_shape=jax.ShapeDtypeStruct((G, OUT), jnp.float32),
    )(p2, t2, m2, b2, batch2d)


@jax.jit
def kernel(x, edge_index, batch, W1, att_src1, att_dst1, b1,
           W2, att_src2, att_dst2, b2):
    x = x.astype(jnp.float32)
    eye = jnp.eye(H, dtype=jnp.float32)
    W1cm = W1.reshape(D, H, C).transpose(0, 2, 1).reshape(D, H * C)
    Ascm = (att_src1.T[:, :, None] * eye[None, :, :]).reshape(C * H, H)
    Adcm = (att_dst1.T[:, :, None] * eye[None, :, :]).reshape(C * H, H)
    Rcm = jnp.tile(eye, (1, C))
    b1cm = b1.reshape(H, C).T.reshape(1, H * C)
    W2cm = W2.reshape(H, C, OUT).transpose(1, 0, 2).reshape(H * C, OUT)
    pad = jnp.full((EPAD - E,), N, jnp.int32)
    srcs = jnp.concatenate([edge_index[0].astype(jnp.int32), pad])
    dsts = jnp.concatenate([edge_index[1].astype(jnp.int32), pad])
    sd = jnp.stack([srcs.reshape(NW, NCH, BE),
                    dsts.reshape(NW, NCH, BE)], axis=2)

    t1, td1, m1 = _pre1(x, W1cm, Ascm, Adcm)
    p1 = _edge1(t1, td1, sd, m1)
    t2, td2, m2 = _mid(p1.reshape(2 * NP, 128), t1, td1, m1,
                       b1cm, W2cm, att_src2.T, att_dst2.T, Rcm)
    p2 = _edge2(t2, td2, sd, m2)
    pooled = _post(p2.reshape(2 * NP, 128), t2, m2, b2[None, :],
                   batch[:, None].astype(jnp.int32))
    return pooled


# speedup vs baseline: 2.7267x; 2.1837x over previous
import functools
import jax
import jax.numpy as jnp
from jax import lax
from jax.experimental import pallas as pl
from jax.experimental.pallas import tpu as pltpu
from jax.experimental.pallas import tpu_sc as plsc

N = 10000
E = 320000
D = 128
H = 8
C = 8
OUT = 16
G = 64

NP = 10112
NW = 32
BE = 72
NCH = 140
NPAIR = NCH // 2
EPAD = NW * NCH * BE
RPW = NP // 16

_mesh = plsc.VectorSubcoreMesh(core_axis_name="c", subcore_axis_name="s")


def _lrelu(x):
    return jnp.maximum(x, 0.2 * x)


def _pre1_body(x_ref, w1_ref, as_ref, ad_ref, t_ref, td_ref, m_ref):
    h = jnp.dot(x_ref[...], w1_ref[...], preferred_element_type=jnp.float32)
    a_s = jnp.dot(h, as_ref[...], preferred_element_type=jnp.float32)
    a_d = jnp.dot(h, ad_ref[...], preferred_element_type=jnp.float32)
    t = jnp.concatenate([h, a_s, a_s,
                         jnp.zeros((N, 48), jnp.float32)], axis=1)
    t_ref[0:N, :] = t
    t_ref[N:NP, :] = jnp.zeros((NP - N, 128), jnp.float32)
    td_ref[0:N, :] = jnp.concatenate([a_d, a_d,
                                      jnp.zeros((N, 112), jnp.float32)], axis=1)
    td_ref[N:NP, :] = jnp.zeros((NP - N, 128), jnp.float32)
    m8 = _lrelu(jnp.max(a_s, axis=0) + jnp.max(a_d, axis=0))
    m_ref[...] = jnp.concatenate([m8, m8])[None, :]


def _pre1(x, W1cm, Ascm, Adcm):
    return pl.pallas_call(
        _pre1_body,
        out_shape=[
            jax.ShapeDtypeStruct((NP, 128), jnp.float32),
            jax.ShapeDtypeStruct((NP, 128), jnp.float32),
            jax.ShapeDtypeStruct((1, 16), jnp.float32),
        ],
    )(x, W1cm, Ascm, Adcm)


def _make_edge_pass(name, a_src_off, nblk):
    aw = 128
    kernel_kwargs = dict(
        mesh=_mesh,
        out_type=jax.ShapeDtypeStruct((2, NP, aw), jnp.float32),
        scratch_types=[
            pltpu.VMEM((2, BE), jnp.int32),
            pltpu.VMEM((2, BE), jnp.int32),
            pltpu.VMEM((BE, 128), jnp.float32),
            pltpu.VMEM((BE, 128), jnp.float32),
            pltpu.VMEM((BE, 128), jnp.float32),
            pltpu.VMEM((BE, 128), jnp.float32),
            pltpu.VMEM((BE, aw), jnp.float32),
            pltpu.VMEM((16,), jnp.float32),
            pltpu.VMEM_SHARED((NP, aw), jnp.float32),
            pltpu.SemaphoreType.DMA,
            pltpu.SemaphoreType.DMA,
        ],
    )

    def _edge(tsrc, tdst, sd, mvec, out, sdA, sdB, srowsA, drowsA, srowsB,
              drowsB, msg, mv, acc, semA, semB):
        c = lax.axis_index("c")
        s = lax.axis_index("s")
        wid = s * 2 + c

        pltpu.sync_copy(mvec.at[0], mv)

        def _zrow(i, _):
            zv = jnp.zeros((16,), jnp.float32)
            for k in range(8):
                msg[i, pl.ds(16 * k, 16)] = zv
            return 0

        lax.fori_loop(0, BE, _zrow, 0)
        base = s * RPW
        for st in list(range(0, RPW - BE, BE)) + [RPW - BE]:
            pltpu.sync_copy(msg, acc.at[pl.ds(base + st, BE)])
        plsc.subcore_barrier()

        def _fire(j, sdb, srows, drows, sem):
            pltpu.sync_copy(sd.at[wid, j], sdb)
            pltpu.async_copy(tsrc.at[sdb.at[0]], srows, sem)
            pltpu.async_copy(tdst.at[sdb.at[1]], drows, sem)

        def _drain(sdb, srows, drows, sem):
            pltpu.make_async_copy(tsrc.at[sdb.at[0]], srows, sem).wait()
            pltpu.make_async_copy(tdst.at[sdb.at[1]], drows, sem).wait()

        def _compute(sdb, srows, drows):
            mreg = mv[...]

            def _edge_i(b, _):
                e = srows[b, pl.ds(a_src_off, 16)] + drows[b, pl.ds(0, 16)]
                w = jnp.exp(_lrelu(e) - mreg)
                for k in range(nblk):
                    msg[b, pl.ds(16 * k, 16)] = srows[b, pl.ds(16 * k, 16)] * w
                msg[b, pl.ds(16 * nblk, 16)] = w
                return 0

            lax.fori_loop(0, BE, _edge_i, 0)
            pltpu.sync_copy(msg, acc.at[sdb.at[1]], add=True)

        _fire(0, sdA, srowsA, drowsA, semA)

        def _pair(jp, _):
            j0 = 2 * jp
            _fire(j0 + 1, sdB, srowsB, drowsB, semB)
            _drain(sdA, srowsA, drowsA, semA)
            _compute(sdA, srowsA, drowsA)
            _fire(jnp.minimum(j0 + 2, NCH - 1), sdA, srowsA, drowsA, semA)
            _drain(sdB, srowsB, drowsB, semB)
            _compute(sdB, srowsB, drowsB)
            return 0

        lax.fori_loop(0, NPAIR, _pair, 0)
        _drain(sdA, srowsA, drowsA, semA)
        plsc.subcore_barrier()
        pltpu.sync_copy(acc.at[pl.ds(base, RPW)], out.at[c, pl.ds(base, RPW)])

    _edge.__name__ = name
    _edge.__qualname__ = name
    return pl.kernel(_edge, **kernel_kwargs)


_edge1 = _make_edge_pass("_edge1", 64, 4)
_edge2 = _make_edge_pass("_edge2", 16, 1)


def _mid_body(p_ref, t1_ref, td1_ref, m1_ref, b1_ref, w2_ref, as2_ref,
              ad2_ref, r_ref, t2_ref, td2_ref, m2_ref):
    Rcm = r_ref[...]
    num = p_ref[0:NP, 0:64] + p_ref[NP:2 * NP, 0:64]
    den8 = p_ref[0:NP, 64:72] + p_ref[NP:2 * NP, 64:72]
    h1 = t1_ref[:, 0:64]
    a_s1 = t1_ref[:, 64:72]
    a_d1 = td1_ref[:, 0:8]
    m1 = m1_ref[0:1, 0:8]
    w_self = jnp.exp(_lrelu(a_s1 + a_d1) - m1)
    num = num + jnp.dot(w_self, Rcm, preferred_element_type=jnp.float32) * h1
    den = jnp.dot(den8 + w_self, Rcm, preferred_element_type=jnp.float32)
    o1 = num / (den + 1e-16) + b1_ref[...]
    o1 = jnp.where(o1 > 0, o1, jnp.exp(jnp.minimum(o1, 0.0)) - 1.0)
    rows = lax.broadcasted_iota(jnp.int32, (NP, 64), 0)
    o1 = jnp.where(rows < N, o1, 0.0)
    h2 = jnp.dot(o1, w2_ref[...], preferred_element_type=jnp.float32)
    asc = jnp.dot(h2, as2_ref[...], preferred_element_type=jnp.float32)
    adc = jnp.dot(h2, ad2_ref[...], preferred_element_type=jnp.float32)
    t2_ref[...] = jnp.concatenate(
        [h2, jnp.broadcast_to(asc, (NP, 16)), jnp.broadcast_to(adc, (NP, 16)),
         jnp.zeros((NP, 80), jnp.float32)], axis=1)
    td2_ref[...] = jnp.concatenate(
        [jnp.broadcast_to(adc, (NP, 16)), jnp.zeros((NP, 112), jnp.float32)],
        axis=1)
    m2 = _lrelu(jnp.max(asc) + jnp.max(adc))
    m2_ref[...] = jnp.full((1, 16), m2, jnp.float32)


def _mid(p1, t1, td1, m1, b1cm, W2cm, as2t, ad2t, Rcm):
    return pl.pallas_call(
        _mid_body,
        out_shape=[
            jax.ShapeDtypeStruct((NP, 128), jnp.float32),
            jax.ShapeDtypeStruct((NP, 128), jnp.float32),
            jax.ShapeDtypeStruct((1, 16), jnp.float32),
        ],
    )(p1, t1, td1, m1, b1cm, W2cm, as2t, ad2t, Rcm)


def _post_body(p_ref, t2_ref, m2_ref, b2_ref, batch_ref, out_ref):
    num2 = p_ref[0:NP, 0:16] + p_ref[NP:2 * NP, 0:16]
    den2 = p_ref[0:NP, 16:17] + p_ref[NP:2 * NP, 16:17]
    h2 = t2_ref[:, 0:16]
    asc = t2_ref[:, 16:17]
    adc = t2_ref[:, 32:33]
    w2 = jnp.exp(_lrelu(asc + adc) - m2_ref[0:1, 0:1])
    numf = num2 + w2 * h2
    denf = den2 + w2
    hout = numf / (denf + 1e-16) + b2_ref[...]
    hN = hout[0:N, :]
    cols = lax.broadcasted_iota(jnp.int32, (N, G), 1)
    onehot = (batch_ref[...] == cols).astype(jnp.float32)
    psum = lax.dot_general(onehot, hN, (((0,), (0,)), ((), ())),
                           preferred_element_type=jnp.float32)
    cnt = lax.dot_general(onehot, jnp.ones((N, 1), jnp.float32),
                          (((0,), (0,)), ((), ())),
                          preferred_element_type=jnp.float32)
    out_ref[...] = psum / jnp.maximum(cnt, 1.0)


def _post(p2, t2, m2, b2, batch2d):
    return pl.pallas_call(
        _post_body,
        out_shape=jax.ShapeDtypeStruct((G, OUT), jnp.float32),
    )(p2, t2, m2, b2, batch2d)


@jax.jit
def kernel(x, edge_index, batch, W1, att_src1, att_dst1, b1,
           W2, att_src2, att_dst2, b2):
    x = x.astype(jnp.float32)
    eye = jnp.eye(H, dtype=jnp.float32)
    W1cm = W1.reshape(D, H, C).transpose(0, 2, 1).reshape(D, H * C)
    Ascm = (att_src1.T[:, :, None] * eye[None, :, :]).reshape(C * H, H)
    Adcm = (att_dst1.T[:, :, None] * eye[None, :, :]).reshape(C * H, H)
    Rcm = jnp.tile(eye, (1, C))
    b1cm = b1.reshape(H, C).T.reshape(1, H * C)
    W2cm = W2.reshape(H, C, OUT).transpose(1, 0, 2).reshape(H * C, OUT)
    pad = jnp.full((EPAD - E,), N, jnp.int32)
    srcs = jnp.concatenate([edge_index[0].astype(jnp.int32), pad])
    dsts = jnp.concatenate([edge_index[1].astype(jnp.int32), pad])
    sd = jnp.stack([srcs.reshape(NW, NCH, BE),
                    dsts.reshape(NW, NCH, BE)], axis=2)

    t1, td1, m1 = _pre1(x, W1cm, Ascm, Adcm)
    p1 = _edge1(t1, td1, sd, m1)
    t2, td2, m2 = _mid(p1.reshape(2 * NP, 128), t1, td1, m1,
                       b1cm, W2cm, att_src2.T, att_dst2.T, Rcm)
    p2 = _edge2(t2, td2, sd, m2)
    pooled = _post(p2.reshape(2 * NP, 128), t2, m2, b2[None, :],
                   batch[:, None].astype(jnp.int32))
    return pooled
